# baseline (speedup 1.0000x reference)
import jax
import jax.numpy as jnp
from jax import lax
from jax.experimental import pallas as pl
from jax.experimental.pallas import tpu as pltpu

E = 8
TOPK = 2
LAYERS = 3
LOSS_COEF = 0.01
D = 768
T = 2048
BT = 512
NBLK = T // BT

_BF = jnp.bfloat16


def _dot(a, b):
    return jnp.dot(a.astype(_BF), b, preferred_element_type=jnp.float32)


def _net_kernel(x_ref, enc_W_ref, enc_b_ref, gate_W_ref, We_ref, be_ref,
                dec_W_ref, dec_b_ref, out_ref, loss_ref, imp_ref):
    i = pl.program_id(0)

    @pl.when(i == 0)
    def _init():
        imp_ref[...] = jnp.zeros_like(imp_ref)

    h = jnp.maximum(_dot(x_ref[:], enc_W_ref[:]) + enc_b_ref[:], 0.0)
    col = lax.broadcasted_iota(jnp.int32, (BT, E), 1)
    for l in range(LAYERS):
        logits = _dot(h, gate_W_ref[:])
        m0 = jnp.max(logits, axis=1, keepdims=True)
        i0 = jnp.min(jnp.where(logits == m0, col, E), axis=1, keepdims=True)
        masked = jnp.where(col == i0, -jnp.inf, logits)
        m1 = jnp.max(masked, axis=1, keepdims=True)
        i1 = jnp.min(jnp.where(masked == m1, col, E), axis=1, keepdims=True)
        u1 = jnp.exp(m1 - m0)
        den = 1.0 + u1
        g0 = 1.0 / den
        g1 = u1 / den
        gates = (jnp.where(col == i0, g0, 0.0)
                 + jnp.where(col == i1, g1, 0.0))
        gates_b = gates.astype(_BF).astype(jnp.float32)
        acc = jnp.zeros((BT, D), jnp.float32)
        for e in range(E):
            he = jnp.maximum(_dot(h, We_ref[e]) + be_ref[e][None, :], 0.0)
            acc = acc + gates_b[:, e:e + 1] * he.astype(_BF).astype(jnp.float32)
        imp_ref[l:l + 1, :] += jnp.sum(gates, axis=0, keepdims=True)
        h = acc
    out_ref[:] = jnp.maximum(_dot(h, dec_W_ref[:]) + dec_b_ref[:], 0.0)

    @pl.when(i == NBLK - 1)
    def _fin():
        loss = jnp.float32(0.0)
        for l in range(LAYERS):
            imp = imp_ref[l:l + 1, :]
            mean = jnp.mean(imp)
            var = jnp.mean((imp - mean) ** 2)
            loss = loss + LOSS_COEF * var / (mean * mean + 1e-10)
        loss_ref[:, :] = jnp.broadcast_to(loss, (1, 1))


@jax.jit
def kernel(x, enc_W, enc_b, gate_W, We, be, dec_W, dec_b):
    full = lambda *s: pl.BlockSpec(s, lambda i: (0,) * len(s))
    out, loss = pl.pallas_call(
        _net_kernel,
        grid=(NBLK,),
        in_specs=[
            pl.BlockSpec((BT, D), lambda i: (i, 0)),
            full(D, D),
            full(1, D),
            full(D, E),
            full(E, D, D),
            full(E, D),
            full(D, D),
            full(1, D),
        ],
        out_specs=(
            pl.BlockSpec((BT, D), lambda i: (i, 0)),
            pl.BlockSpec((1, 1), lambda i: (0, 0)),
        ),
        out_shape=(
            jax.ShapeDtypeStruct((T, D), jnp.float32),
            jax.ShapeDtypeStruct((1, 1), jnp.float32),
        ),
        scratch_shapes=[pltpu.VMEM((LAYERS, E), jnp.float32)],
        compiler_params=pltpu.CompilerParams(
            dimension_semantics=("arbitrary",),
            vmem_limit_bytes=100 * 1024 * 1024,
        ),
    )(x, enc_W.astype(_BF), enc_b.reshape(1, D), gate_W.astype(_BF),
      We.astype(_BF), be, dec_W.astype(_BF), dec_b.reshape(1, D))
    return out, loss.reshape(())


# speedup vs baseline: 1.1064x; 1.1064x over previous
import jax
import jax.numpy as jnp
from jax import lax
from jax.experimental import pallas as pl
from jax.experimental.pallas import tpu as pltpu

E = 8
TOPK = 2
LAYERS = 3
LOSS_COEF = 0.01
D = 768
T = 2048
BT = 1024
NBLK = T // BT

_BF = jnp.bfloat16


def _dot(a, b):
    return jnp.dot(a.astype(_BF), b.astype(_BF),
                   preferred_element_type=jnp.float32)


def _net_kernel(x_ref, enc_W_ref, enc_b_ref, gate_W_ref, We_ref, be_ref,
                dec_W_ref, dec_b_ref, out_ref, loss_ref, imp_ref):
    i = pl.program_id(0)

    @pl.when(i == 0)
    def _init():
        imp_ref[...] = jnp.zeros_like(imp_ref)

    h = jnp.maximum(_dot(x_ref[:], enc_W_ref[:]) + enc_b_ref[:], 0.0)
    col = lax.broadcasted_iota(jnp.int32, (BT, E), 1)
    for l in range(LAYERS):
        logits = _dot(h, gate_W_ref[:])
        m0 = jnp.max(logits, axis=1, keepdims=True)
        i0 = jnp.min(jnp.where(logits == m0, col, E), axis=1, keepdims=True)
        masked = jnp.where(col == i0, -jnp.inf, logits)
        m1 = jnp.max(masked, axis=1, keepdims=True)
        i1 = jnp.min(jnp.where(masked == m1, col, E), axis=1, keepdims=True)
        u1 = jnp.exp(m1 - m0)
        den = 1.0 + u1
        g0 = 1.0 / den
        g1 = u1 / den
        gates = (jnp.where(col == i0, g0, 0.0)
                 + jnp.where(col == i1, g1, 0.0))
        gates_b = gates.astype(_BF).astype(jnp.float32)
        acc = jnp.zeros((BT, D), jnp.float32)
        for e in range(E):
            he = jnp.maximum(_dot(h, We_ref[e]) + be_ref[e][None, :], 0.0)
            acc = acc + gates_b[:, e:e + 1] * he.astype(_BF).astype(jnp.float32)
        imp_ref[l:l + 1, :] += jnp.sum(gates, axis=0, keepdims=True)
        h = acc
    out_ref[:] = jnp.maximum(_dot(h, dec_W_ref[:]) + dec_b_ref[:], 0.0)

    @pl.when(i == NBLK - 1)
    def _fin():
        loss = jnp.float32(0.0)
        for l in range(LAYERS):
            imp = imp_ref[l:l + 1, :]
            mean = jnp.mean(imp)
            var = jnp.mean((imp - mean) ** 2)
            loss = loss + LOSS_COEF * var / (mean * mean + 1e-10)
        loss_ref[:, :] = jnp.broadcast_to(loss, (1, 1))


@jax.jit
def kernel(x, enc_W, enc_b, gate_W, We, be, dec_W, dec_b):
    full = lambda *s: pl.BlockSpec(s, lambda i: (0,) * len(s))
    out, loss = pl.pallas_call(
        _net_kernel,
        grid=(NBLK,),
        in_specs=[
            pl.BlockSpec((BT, D), lambda i: (i, 0)),
            full(D, D),
            full(1, D),
            full(D, E),
            full(E, D, D),
            full(E, D),
            full(D, D),
            full(1, D),
        ],
        out_specs=(
            pl.BlockSpec((BT, D), lambda i: (i, 0)),
            pl.BlockSpec((1, 1), lambda i: (0, 0)),
        ),
        out_shape=(
            jax.ShapeDtypeStruct((T, D), jnp.float32),
            jax.ShapeDtypeStruct((1, 1), jnp.float32),
        ),
        scratch_shapes=[pltpu.VMEM((LAYERS, E), jnp.float32)],
        compiler_params=pltpu.CompilerParams(
            dimension_semantics=("arbitrary",),
            vmem_limit_bytes=100 * 1024 * 1024,
        ),
    )(x, enc_W, enc_b.reshape(1, D), gate_W, We, be,
      dec_W, dec_b.reshape(1, D))
    return out, loss.reshape(())
